# baseline (speedup 1.0000x reference)
import functools

import jax
import jax.numpy as jnp
import numpy as np
from jax import lax
from jax.experimental import pallas as pl
from jax.experimental.pallas import tpu as pltpu
from jax.experimental.pallas import tpu_sc as plsc

B = 256
TS_LEN = 1024
SEG_LEN = 8
DIM_SEQ = 128
DIM_MODEL = 768
N_BUCKETS = 128
WORDS = 64
NSEG = B * DIM_SEQ
VOCAB = N_BUCKETS * WORDS


def _build_haar_w():
    a = np.eye(TS_LEN, dtype=np.float64)
    details = []
    while a.shape[1] > 1:
        d = (a[:, 0::2] - a[:, 1::2]) / np.sqrt(2.0)
        a = (a[:, 0::2] + a[:, 1::2]) / np.sqrt(2.0)
        details.append(d)
    coeffs = ([a] + details[::-1])[: SEG_LEN]
    rows = [np.repeat(c, DIM_SEQ // c.shape[1], axis=1) for c in coeffs]
    w = np.stack(rows, axis=2).reshape(TS_LEN, DIM_SEQ * SEG_LEN)
    return w.astype(np.float32)


_HAAR_W = _build_haar_w()


def _coef_body(x_ref, w_ref, coef_ref):
    coef_ref[...] = jnp.dot(x_ref[...], w_ref[...],
                            preferred_element_type=jnp.float32,
                            precision=lax.Precision.HIGHEST)


def _tc_coeffs(x):
    return pl.pallas_call(
        _coef_body,
        grid=(4,),
        in_specs=[
            pl.BlockSpec((B // 4, TS_LEN), lambda i: (i, 0)),
            pl.BlockSpec((TS_LEN, TS_LEN), lambda i: (0, 0)),
        ],
        out_specs=pl.BlockSpec((B // 4, TS_LEN), lambda i: (i, 0)),
        out_shape=jax.ShapeDtypeStruct((B, TS_LEN), jnp.float32),
    )(x, _HAAR_W)


def _idx_body(xseg_ref, vf_ref, idx_ref):
    segs = xseg_ref[...]
    diffs = segs[:, 1:] - segs[:, :-1]
    e7 = lax.broadcasted_iota(jnp.int32, (diffs.shape[0], 7), 1)
    bits = jnp.where(diffs > 0, jnp.int32(1) << e7, 0)
    bucket = jnp.sum(bits, axis=1, keepdims=True)
    smin = jnp.min(segs, axis=1, keepdims=True)
    smax = jnp.max(segs, axis=1, keepdims=True)
    narr = (segs - smin) / (smax - smin + 1e-08)
    lanes = lax.broadcasted_iota(jnp.int32, (segs.shape[0], N_BUCKETS), 1)
    oh = (bucket == lanes).astype(jnp.float32)
    bw = jnp.dot(oh, vf_ref[...], preferred_element_type=jnp.float32,
                 precision=lax.Precision.HIGHEST)
    d = [jnp.abs(bw[:, e * WORDS:(e + 1) * WORDS] - narr[:, e:e + 1])
         for e in range(SEG_LEN)]
    dist = ((d[0] + d[1]) + (d[2] + d[3])) + ((d[4] + d[5]) + (d[6] + d[7]))
    best = jnp.argmin(dist, axis=1).astype(jnp.int32)
    idx_ref[...] = bucket * WORDS + best[:, None]


def _tc_index(xseg, vf):
    grid = 16
    sb = NSEG // grid
    return pl.pallas_call(
        _idx_body,
        grid=(grid,),
        in_specs=[
            pl.BlockSpec((sb, SEG_LEN), lambda i: (i, 0)),
            pl.BlockSpec((N_BUCKETS, SEG_LEN * WORDS), lambda i: (0, 0)),
        ],
        out_specs=pl.BlockSpec((sb, 1), lambda i: (i, 0)),
        out_shape=jax.ShapeDtypeStruct((NSEG, 1), jnp.int32),
    )(xseg, vf)


_NC, _NS = 2, 16
_NW = _NC * _NS
_BPW = NSEG // _NW
_CHUNK = 64
_NCH = _BPW // _CHUNK


@functools.cache
def _make_sc_gather(nrows):
    bpw = nrows // _NW
    nch = bpw // _CHUNK

    @functools.partial(
        pl.kernel,
        mesh=plsc.VectorSubcoreMesh(core_axis_name="c", subcore_axis_name="s"),
        out_type=jax.ShapeDtypeStruct((nrows, DIM_MODEL), jnp.float32),
        scratch_types=[
            pltpu.VMEM((bpw,), jnp.int32),
            pltpu.VMEM((_CHUNK, DIM_MODEL), jnp.float32),
            pltpu.VMEM((_CHUNK, DIM_MODEL), jnp.float32),
            pltpu.SemaphoreType.DMA,
            pltpu.SemaphoreType.DMA,
            pltpu.SemaphoreType.DMA,
            pltpu.SemaphoreType.DMA,
        ],
    )
    def _sc_gather(table_hbm, idx_hbm, out_hbm, idx_v, buf0, buf1,
                   gsem0, gsem1, ssem0, ssem1):
        wid = lax.axis_index("s") * _NC + lax.axis_index("c")
        base = pl.multiple_of(wid * bpw, _CHUNK)
        bufs = (buf0, buf1)
        gsems = (gsem0, gsem1)
        ssems = (ssem0, ssem1)
        pltpu.sync_copy(idx_hbm.at[pl.ds(base, bpw)], idx_v)

        def gather(j):
            idx_c = idx_v.at[pl.ds(j * _CHUNK, _CHUNK)]
            return pltpu.async_copy(table_hbm.at[idx_c], bufs[j % 2],
                                    gsems[j % 2])

        def store(j):
            return pltpu.async_copy(
                bufs[j % 2], out_hbm.at[pl.ds(base + j * _CHUNK, _CHUNK)],
                ssems[j % 2])

        gathers = [None] * nch
        stores = [None] * nch
        gathers[0] = gather(0)
        for j in range(nch):
            gathers[j].wait()
            stores[j] = store(j)
            if j + 1 < nch:
                if j - 1 >= 0:
                    stores[j - 1].wait()
                gathers[j + 1] = gather(j + 1)
        stores[nch - 2].wait()
        stores[nch - 1].wait()

    return _sc_gather


_BROWS = 8


def _combine_body(seq_ref, c_ref, he_ref, pos_ref, out_ref):
    he = he_ref[...]
    pos = pos_ref[...]
    for b in range(_BROWS):
        hp = jnp.dot(c_ref[b], he, preferred_element_type=jnp.float32)
        out_ref[b] = seq_ref[b] + hp + pos


_HB = B // 2


def _combine_body2(prev_ref, seq_ref, c_ref, he_ref, pos_ref, out_ref):
    del prev_ref
    _combine_body(seq_ref, c_ref, he_ref, pos_ref, out_ref)


def _tc_combine_first(seq3, coeffs3, haar_emb, pos_emb):
    return pl.pallas_call(
        _combine_body,
        grid=(_HB // _BROWS,),
        in_specs=[
            pl.BlockSpec((_BROWS, DIM_SEQ, DIM_MODEL), lambda i: (i, 0, 0)),
            pl.BlockSpec((_BROWS, DIM_SEQ, SEG_LEN), lambda i: (i, 0, 0)),
            pl.BlockSpec((SEG_LEN, DIM_MODEL), lambda i: (0, 0)),
            pl.BlockSpec((DIM_SEQ, DIM_MODEL), lambda i: (0, 0)),
        ],
        out_specs=pl.BlockSpec((_BROWS, DIM_SEQ, DIM_MODEL),
                               lambda i: (i, 0, 0)),
        out_shape=jax.ShapeDtypeStruct((B, DIM_SEQ, DIM_MODEL), jnp.float32),
    )(seq3, coeffs3, haar_emb, pos_emb)


def _tc_combine_second(prev, seq3, coeffs3, haar_emb, pos_emb):
    hoff = _HB // _BROWS
    return pl.pallas_call(
        _combine_body2,
        grid=(_HB // _BROWS,),
        in_specs=[
            pl.BlockSpec(memory_space=pl.ANY),
            pl.BlockSpec((_BROWS, DIM_SEQ, DIM_MODEL), lambda i: (i, 0, 0)),
            pl.BlockSpec((_BROWS, DIM_SEQ, SEG_LEN), lambda i: (i, 0, 0)),
            pl.BlockSpec((SEG_LEN, DIM_MODEL), lambda i: (0, 0)),
            pl.BlockSpec((DIM_SEQ, DIM_MODEL), lambda i: (0, 0)),
        ],
        out_specs=pl.BlockSpec((_BROWS, DIM_SEQ, DIM_MODEL),
                               lambda i: (i + hoff, 0, 0)),
        out_shape=jax.ShapeDtypeStruct((B, DIM_SEQ, DIM_MODEL), jnp.float32),
        input_output_aliases={0: 0},
    )(prev, seq3, coeffs3, haar_emb, pos_emb)


def kernel(X, vocab_words, word_emb, haar_emb, pos_emb):
    xseg = X.reshape(NSEG, SEG_LEN)
    vf = vocab_words.transpose(0, 2, 1).reshape(N_BUCKETS, SEG_LEN * WORDS)
    idx2 = _tc_index(xseg, vf)
    idxf = idx2.reshape(NSEG)
    half = NSEG // 2
    gather = _make_sc_gather(half)
    seq1 = gather(word_emb, idxf[:half])
    seq2 = gather(word_emb, idxf[half:])
    coeffs = _tc_coeffs(X)
    c3 = coeffs.reshape(B, DIM_SEQ, SEG_LEN)
    out1 = _tc_combine_first(seq1.reshape(_HB, DIM_SEQ, DIM_MODEL),
                             c3[:_HB], haar_emb, pos_emb)
    out = _tc_combine_second(out1, seq2.reshape(_HB, DIM_SEQ, DIM_MODEL),
                             c3[_HB:], haar_emb, pos_emb)
    att_mask = jnp.ones((B, DIM_SEQ), dtype=jnp.int32)
    return (out, att_mask)


# speedup vs baseline: 1.0345x; 1.0345x over previous
import functools

import jax
import jax.numpy as jnp
import numpy as np
from jax import lax
from jax.experimental import pallas as pl
from jax.experimental.pallas import tpu as pltpu
from jax.experimental.pallas import tpu_sc as plsc

B = 256
TS_LEN = 1024
SEG_LEN = 8
DIM_SEQ = 128
DIM_MODEL = 768
N_BUCKETS = 128
WORDS = 64
NSEG = B * DIM_SEQ
VOCAB = N_BUCKETS * WORDS


def _build_haar_w():
    a = np.eye(TS_LEN, dtype=np.float64)
    details = []
    while a.shape[1] > 1:
        d = (a[:, 0::2] - a[:, 1::2]) / np.sqrt(2.0)
        a = (a[:, 0::2] + a[:, 1::2]) / np.sqrt(2.0)
        details.append(d)
    coeffs = ([a] + details[::-1])[: SEG_LEN]
    rows = [np.repeat(c, DIM_SEQ // c.shape[1], axis=1) for c in coeffs]
    w = np.stack(rows, axis=2).reshape(TS_LEN, DIM_SEQ * SEG_LEN)
    return w.astype(np.float32)


_HAAR_W = _build_haar_w()


def _coef_body(x_ref, w_ref, coef_ref):
    coef_ref[...] = jnp.dot(x_ref[...], w_ref[...],
                            preferred_element_type=jnp.float32,
                            precision=lax.Precision.HIGHEST)


def _tc_coeffs(x):
    return pl.pallas_call(
        _coef_body,
        grid=(4,),
        in_specs=[
            pl.BlockSpec((B // 4, TS_LEN), lambda i: (i, 0)),
            pl.BlockSpec((TS_LEN, TS_LEN), lambda i: (0, 0)),
        ],
        out_specs=pl.BlockSpec((B // 4, TS_LEN), lambda i: (i, 0)),
        out_shape=jax.ShapeDtypeStruct((B, TS_LEN), jnp.float32),
    )(x, _HAAR_W)


def _idx_body(xseg_ref, vf_ref, idx_ref):
    segs = xseg_ref[...]
    diffs = segs[:, 1:] - segs[:, :-1]
    e7 = lax.broadcasted_iota(jnp.int32, (diffs.shape[0], 7), 1)
    bits = jnp.where(diffs > 0, jnp.int32(1) << e7, 0)
    bucket = jnp.sum(bits, axis=1, keepdims=True)
    smin = jnp.min(segs, axis=1, keepdims=True)
    smax = jnp.max(segs, axis=1, keepdims=True)
    narr = (segs - smin) / (smax - smin + 1e-08)
    lanes = lax.broadcasted_iota(jnp.int32, (segs.shape[0], N_BUCKETS), 1)
    oh = (bucket == lanes).astype(jnp.float32)
    bw = jnp.dot(oh, vf_ref[...], preferred_element_type=jnp.float32,
                 precision=lax.Precision.HIGHEST)
    d = [jnp.abs(bw[:, e * WORDS:(e + 1) * WORDS] - narr[:, e:e + 1])
         for e in range(SEG_LEN)]
    dist = ((d[0] + d[1]) + (d[2] + d[3])) + ((d[4] + d[5]) + (d[6] + d[7]))
    best = jnp.argmin(dist, axis=1).astype(jnp.int32)
    idx_ref[...] = bucket * WORDS + best[:, None]


def _tc_index(xseg, vf):
    nseg = xseg.shape[0]
    sb = 2048
    return pl.pallas_call(
        _idx_body,
        grid=(nseg // sb,),
        in_specs=[
            pl.BlockSpec((sb, SEG_LEN), lambda i: (i, 0)),
            pl.BlockSpec((N_BUCKETS, SEG_LEN * WORDS), lambda i: (0, 0)),
        ],
        out_specs=pl.BlockSpec((sb, 1), lambda i: (i, 0)),
        out_shape=jax.ShapeDtypeStruct((nseg, 1), jnp.int32),
    )(xseg, vf)


_NC, _NS = 2, 16
_NW = _NC * _NS
_BPW = NSEG // _NW
_CHUNK = 64
_NCH = _BPW // _CHUNK


@functools.cache
def _make_sc_gather(nrows):
    bpw = nrows // _NW
    nch = bpw // _CHUNK

    @functools.partial(
        pl.kernel,
        mesh=plsc.VectorSubcoreMesh(core_axis_name="c", subcore_axis_name="s"),
        out_type=jax.ShapeDtypeStruct((nrows, DIM_MODEL), jnp.float32),
        scratch_types=[
            pltpu.VMEM((bpw,), jnp.int32),
            pltpu.VMEM((_CHUNK, DIM_MODEL), jnp.float32),
            pltpu.VMEM((_CHUNK, DIM_MODEL), jnp.float32),
            pltpu.SemaphoreType.DMA,
            pltpu.SemaphoreType.DMA,
            pltpu.SemaphoreType.DMA,
            pltpu.SemaphoreType.DMA,
        ],
    )
    def _sc_gather(table_hbm, idx_hbm, out_hbm, idx_v, buf0, buf1,
                   gsem0, gsem1, ssem0, ssem1):
        wid = lax.axis_index("s") * _NC + lax.axis_index("c")
        base = pl.multiple_of(wid * bpw, _CHUNK)
        bufs = (buf0, buf1)
        gsems = (gsem0, gsem1)
        ssems = (ssem0, ssem1)
        pltpu.sync_copy(idx_hbm.at[pl.ds(base, bpw)], idx_v)

        def gather(j):
            idx_c = idx_v.at[pl.ds(j * _CHUNK, _CHUNK)]
            return pltpu.async_copy(table_hbm.at[idx_c], bufs[j % 2],
                                    gsems[j % 2])

        def store(j):
            return pltpu.async_copy(
                bufs[j % 2], out_hbm.at[pl.ds(base + j * _CHUNK, _CHUNK)],
                ssems[j % 2])

        gathers = [None] * nch
        stores = [None] * nch
        gathers[0] = gather(0)
        for j in range(nch):
            gathers[j].wait()
            stores[j] = store(j)
            if j + 1 < nch:
                if j - 1 >= 0:
                    stores[j - 1].wait()
                gathers[j + 1] = gather(j + 1)
        stores[nch - 2].wait()
        stores[nch - 1].wait()

    return _sc_gather


_BROWS = 8


def _combine_body(seq_ref, c_ref, he_ref, pos_ref, out_ref):
    he = he_ref[...]
    pos = pos_ref[...]
    for b in range(_BROWS):
        hp = jnp.dot(c_ref[b], he, preferred_element_type=jnp.float32)
        out_ref[b] = seq_ref[b] + hp + pos


_HB = B // 2


def _combine_body2(prev_ref, seq_ref, c_ref, he_ref, pos_ref, out_ref):
    del prev_ref
    _combine_body(seq_ref, c_ref, he_ref, pos_ref, out_ref)


def _tc_combine_first(seq3, coeffs3, haar_emb, pos_emb):
    return pl.pallas_call(
        _combine_body,
        grid=(_HB // _BROWS,),
        in_specs=[
            pl.BlockSpec((_BROWS, DIM_SEQ, DIM_MODEL), lambda i: (i, 0, 0)),
            pl.BlockSpec((_BROWS, DIM_SEQ, SEG_LEN), lambda i: (i, 0, 0)),
            pl.BlockSpec((SEG_LEN, DIM_MODEL), lambda i: (0, 0)),
            pl.BlockSpec((DIM_SEQ, DIM_MODEL), lambda i: (0, 0)),
        ],
        out_specs=pl.BlockSpec((_BROWS, DIM_SEQ, DIM_MODEL),
                               lambda i: (i, 0, 0)),
        out_shape=jax.ShapeDtypeStruct((B, DIM_SEQ, DIM_MODEL), jnp.float32),
    )(seq3, coeffs3, haar_emb, pos_emb)


def _tc_combine_second(prev, seq3, coeffs3, haar_emb, pos_emb):
    hoff = _HB // _BROWS
    return pl.pallas_call(
        _combine_body2,
        grid=(_HB // _BROWS,),
        in_specs=[
            pl.BlockSpec(memory_space=pl.ANY),
            pl.BlockSpec((_BROWS, DIM_SEQ, DIM_MODEL), lambda i: (i, 0, 0)),
            pl.BlockSpec((_BROWS, DIM_SEQ, SEG_LEN), lambda i: (i, 0, 0)),
            pl.BlockSpec((SEG_LEN, DIM_MODEL), lambda i: (0, 0)),
            pl.BlockSpec((DIM_SEQ, DIM_MODEL), lambda i: (0, 0)),
        ],
        out_specs=pl.BlockSpec((_BROWS, DIM_SEQ, DIM_MODEL),
                               lambda i: (i + hoff, 0, 0)),
        out_shape=jax.ShapeDtypeStruct((B, DIM_SEQ, DIM_MODEL), jnp.float32),
        input_output_aliases={0: 0},
    )(prev, seq3, coeffs3, haar_emb, pos_emb)


def kernel(X, vocab_words, word_emb, haar_emb, pos_emb):
    xseg = X.reshape(NSEG, SEG_LEN)
    vf = vocab_words.transpose(0, 2, 1).reshape(N_BUCKETS, SEG_LEN * WORDS)
    half = NSEG // 2
    gather = _make_sc_gather(half)
    idx_a = _tc_index(xseg[:half], vf)
    seq1 = gather(word_emb, idx_a.reshape(half))
    idx_b = _tc_index(xseg[half:], vf)
    seq2 = gather(word_emb, idx_b.reshape(half))
    coeffs = _tc_coeffs(X)
    c3 = coeffs.reshape(B, DIM_SEQ, SEG_LEN)
    out1 = _tc_combine_first(seq1.reshape(_HB, DIM_SEQ, DIM_MODEL),
                             c3[:_HB], haar_emb, pos_emb)
    out = _tc_combine_second(out1, seq2.reshape(_HB, DIM_SEQ, DIM_MODEL),
                             c3[_HB:], haar_emb, pos_emb)
    att_mask = jnp.ones((B, DIM_SEQ), dtype=jnp.int32)
    return (out, att_mask)


# speedup vs baseline: 1.0464x; 1.0116x over previous
import functools

import jax
import jax.numpy as jnp
import numpy as np
from jax import lax
from jax.experimental import pallas as pl
from jax.experimental.pallas import tpu as pltpu
from jax.experimental.pallas import tpu_sc as plsc

B = 256
TS_LEN = 1024
SEG_LEN = 8
DIM_SEQ = 128
DIM_MODEL = 768
N_BUCKETS = 128
WORDS = 64
NSEG = B * DIM_SEQ
VOCAB = N_BUCKETS * WORDS


def _build_haar_w():
    a = np.eye(TS_LEN, dtype=np.float64)
    details = []
    while a.shape[1] > 1:
        d = (a[:, 0::2] - a[:, 1::2]) / np.sqrt(2.0)
        a = (a[:, 0::2] + a[:, 1::2]) / np.sqrt(2.0)
        details.append(d)
    coeffs = ([a] + details[::-1])[: SEG_LEN]
    rows = [np.repeat(c, DIM_SEQ // c.shape[1], axis=1) for c in coeffs]
    w = np.stack(rows, axis=2).reshape(TS_LEN, DIM_SEQ * SEG_LEN)
    return w.astype(np.float32)


_HAAR_W = _build_haar_w()


def _coef_body(x_ref, w_ref, coef_ref):
    coef_ref[...] = jnp.dot(x_ref[...], w_ref[...],
                            preferred_element_type=jnp.float32,
                            precision=lax.Precision.HIGHEST)


def _tc_coeffs(x):
    return pl.pallas_call(
        _coef_body,
        grid=(4,),
        in_specs=[
            pl.BlockSpec((B // 4, TS_LEN), lambda i: (i, 0)),
            pl.BlockSpec((TS_LEN, TS_LEN), lambda i: (0, 0)),
        ],
        out_specs=pl.BlockSpec((B // 4, TS_LEN), lambda i: (i, 0)),
        out_shape=jax.ShapeDtypeStruct((B, TS_LEN), jnp.float32),
    )(x, _HAAR_W)


def _idx_body(xseg_ref, vf_ref, idx_ref):
    segs = xseg_ref[...]
    diffs = segs[:, 1:] - segs[:, :-1]
    e7 = lax.broadcasted_iota(jnp.int32, (diffs.shape[0], 7), 1)
    bits = jnp.where(diffs > 0, jnp.int32(1) << e7, 0)
    bucket = jnp.sum(bits, axis=1, keepdims=True)
    smin = jnp.min(segs, axis=1, keepdims=True)
    smax = jnp.max(segs, axis=1, keepdims=True)
    narr = (segs - smin) / (smax - smin + 1e-08)
    lanes = lax.broadcasted_iota(jnp.int32, (segs.shape[0], N_BUCKETS), 1)
    oh = (bucket == lanes).astype(jnp.bfloat16)
    nw = SEG_LEN * WORDS
    bw3 = jnp.dot(oh, vf_ref[...], preferred_element_type=jnp.float32)
    bw = (bw3[:, :nw] + bw3[:, nw:2 * nw]) + bw3[:, 2 * nw:]
    d = [jnp.abs(bw[:, e * WORDS:(e + 1) * WORDS] - narr[:, e:e + 1])
         for e in range(SEG_LEN)]
    dist = ((d[0] + d[1]) + (d[2] + d[3])) + ((d[4] + d[5]) + (d[6] + d[7]))
    best = jnp.argmin(dist, axis=1).astype(jnp.int32)
    idx_ref[...] = bucket * WORDS + best[:, None]


def _split_limbs3(v):
    hi = v.astype(jnp.bfloat16)
    r1 = v - hi.astype(jnp.float32)
    mid = r1.astype(jnp.bfloat16)
    lo = (r1 - mid.astype(jnp.float32)).astype(jnp.bfloat16)
    return hi, mid, lo


def _tc_index(xseg, vf_limbs):
    nseg = xseg.shape[0]
    sb = 2048
    return pl.pallas_call(
        _idx_body,
        grid=(nseg // sb,),
        in_specs=[
            pl.BlockSpec((sb, SEG_LEN), lambda i: (i, 0)),
            pl.BlockSpec((N_BUCKETS, 3 * SEG_LEN * WORDS), lambda i: (0, 0)),
        ],
        out_specs=pl.BlockSpec((sb, 1), lambda i: (i, 0)),
        out_shape=jax.ShapeDtypeStruct((nseg, 1), jnp.int32),
    )(xseg, vf_limbs)


_NC, _NS = 2, 16
_NW = _NC * _NS
_BPW = NSEG // _NW
_CHUNK = 64
_NCH = _BPW // _CHUNK


@functools.cache
def _make_sc_gather(nrows):
    bpw = nrows // _NW
    nch = bpw // _CHUNK

    @functools.partial(
        pl.kernel,
        mesh=plsc.VectorSubcoreMesh(core_axis_name="c", subcore_axis_name="s"),
        out_type=jax.ShapeDtypeStruct((nrows, DIM_MODEL), jnp.float32),
        scratch_types=[
            pltpu.VMEM((bpw,), jnp.int32),
            pltpu.VMEM((_CHUNK, DIM_MODEL), jnp.float32),
            pltpu.VMEM((_CHUNK, DIM_MODEL), jnp.float32),
            pltpu.SemaphoreType.DMA,
            pltpu.SemaphoreType.DMA,
            pltpu.SemaphoreType.DMA,
            pltpu.SemaphoreType.DMA,
        ],
    )
    def _sc_gather(table_hbm, idx_hbm, out_hbm, idx_v, buf0, buf1,
                   gsem0, gsem1, ssem0, ssem1):
        wid = lax.axis_index("s") * _NC + lax.axis_index("c")
        base = pl.multiple_of(wid * bpw, _CHUNK)
        bufs = (buf0, buf1)
        gsems = (gsem0, gsem1)
        ssems = (ssem0, ssem1)
        pltpu.sync_copy(idx_hbm.at[pl.ds(base, bpw)], idx_v)

        def gather(j):
            idx_c = idx_v.at[pl.ds(j * _CHUNK, _CHUNK)]
            return pltpu.async_copy(table_hbm.at[idx_c], bufs[j % 2],
                                    gsems[j % 2])

        def store(j):
            return pltpu.async_copy(
                bufs[j % 2], out_hbm.at[pl.ds(base + j * _CHUNK, _CHUNK)],
                ssems[j % 2])

        gathers = [None] * nch
        stores = [None] * nch
        gathers[0] = gather(0)
        for j in range(nch):
            gathers[j].wait()
            stores[j] = store(j)
            if j + 1 < nch:
                if j - 1 >= 0:
                    stores[j - 1].wait()
                gathers[j + 1] = gather(j + 1)
        stores[nch - 2].wait()
        stores[nch - 1].wait()

    return _sc_gather


_BROWS = 8


def _combine_body(seq_ref, c_ref, he_ref, pos_ref, out_ref):
    he = he_ref[...]
    pos = pos_ref[...]
    for b in range(_BROWS):
        hp = jnp.dot(c_ref[b], he, preferred_element_type=jnp.float32)
        out_ref[b] = seq_ref[b] + hp + pos


_HB = B // 2


def _combine_body2(prev_ref, seq_ref, c_ref, he_ref, pos_ref, out_ref):
    del prev_ref
    _combine_body(seq_ref, c_ref, he_ref, pos_ref, out_ref)


def _tc_combine_first(seq3, coeffs3, haar_emb, pos_emb):
    return pl.pallas_call(
        _combine_body,
        grid=(_HB // _BROWS,),
        in_specs=[
            pl.BlockSpec((_BROWS, DIM_SEQ, DIM_MODEL), lambda i: (i, 0, 0)),
            pl.BlockSpec((_BROWS, DIM_SEQ, SEG_LEN), lambda i: (i, 0, 0)),
            pl.BlockSpec((SEG_LEN, DIM_MODEL), lambda i: (0, 0)),
            pl.BlockSpec((DIM_SEQ, DIM_MODEL), lambda i: (0, 0)),
        ],
        out_specs=pl.BlockSpec((_BROWS, DIM_SEQ, DIM_MODEL),
                               lambda i: (i, 0, 0)),
        out_shape=jax.ShapeDtypeStruct((B, DIM_SEQ, DIM_MODEL), jnp.float32),
    )(seq3, coeffs3, haar_emb, pos_emb)


def _tc_combine_second(prev, seq3, coeffs3, haar_emb, pos_emb):
    hoff = _HB // _BROWS
    return pl.pallas_call(
        _combine_body2,
        grid=(_HB // _BROWS,),
        in_specs=[
            pl.BlockSpec(memory_space=pl.ANY),
            pl.BlockSpec((_BROWS, DIM_SEQ, DIM_MODEL), lambda i: (i, 0, 0)),
            pl.BlockSpec((_BROWS, DIM_SEQ, SEG_LEN), lambda i: (i, 0, 0)),
            pl.BlockSpec((SEG_LEN, DIM_MODEL), lambda i: (0, 0)),
            pl.BlockSpec((DIM_SEQ, DIM_MODEL), lambda i: (0, 0)),
        ],
        out_specs=pl.BlockSpec((_BROWS, DIM_SEQ, DIM_MODEL),
                               lambda i: (i + hoff, 0, 0)),
        out_shape=jax.ShapeDtypeStruct((B, DIM_SEQ, DIM_MODEL), jnp.float32),
        input_output_aliases={0: 0},
    )(prev, seq3, coeffs3, haar_emb, pos_emb)


def kernel(X, vocab_words, word_emb, haar_emb, pos_emb):
    xseg = X.reshape(NSEG, SEG_LEN)
    vf = vocab_words.transpose(0, 2, 1).reshape(N_BUCKETS, SEG_LEN * WORDS)
    vf_limbs = jnp.concatenate(_split_limbs3(vf), axis=1)
    half = NSEG // 2
    gather = _make_sc_gather(half)
    idx_a = _tc_index(xseg[:half], vf_limbs)
    seq1 = gather(word_emb, idx_a.reshape(half))
    idx_b = _tc_index(xseg[half:], vf_limbs)
    seq2 = gather(word_emb, idx_b.reshape(half))
    coeffs = _tc_coeffs(X)
    c3 = coeffs.reshape(B, DIM_SEQ, SEG_LEN)
    out1 = _tc_combine_first(seq1.reshape(_HB, DIM_SEQ, DIM_MODEL),
                             c3[:_HB], haar_emb, pos_emb)
    out = _tc_combine_second(out1, seq2.reshape(_HB, DIM_SEQ, DIM_MODEL),
                             c3[_HB:], haar_emb, pos_emb)
    att_mask = jnp.ones((B, DIM_SEQ), dtype=jnp.int32)
    return (out, att_mask)
